# baseline (device time: 48989 ns/iter reference)
import jax
import jax.numpy as jnp
from jax import lax
from jax.experimental import pallas as pl
from jax.experimental.pallas import tpu as pltpu

N_DEV = 4
B_LOC = 2
H_LOC = 4
SQ = 128
DH = 64
SCALE = 0.125


def kernel(x, Wq, K_ext, V_ext, Wo):
    my = lax.axis_index("i")
    K_my = lax.dynamic_slice_in_dim(K_ext, my * B_LOC, B_LOC, axis=0)
    V_my = lax.dynamic_slice_in_dim(V_ext, my * B_LOC, B_LOC, axis=0)
    K_t = jnp.transpose(K_my, (0, 2, 1, 3))
    V_t = jnp.transpose(V_my, (0, 2, 1, 3))

    def body(x_ref, wq_ref, kt_ref, vt_ref, wo_ref, out_ref,
             comm_wq, comm_wo, send_sems, recv_sems):
        me = lax.axis_index("i")
        right = lax.rem(me + 1, N_DEV)
        left = lax.rem(me + N_DEV - 1, N_DEV)

        barrier_sem = pltpu.get_barrier_semaphore()
        for nbr in (left, right):
            pl.semaphore_signal(
                barrier_sem, inc=1,
                device_id=(nbr,), device_id_type=pl.DeviceIdType.MESH,
            )
        pl.semaphore_wait(barrier_sem, 2)

        comm_wq[0] = wq_ref[...]
        comm_wo[0] = wo_ref[...]

        for h in range(N_DEV):
            if h < N_DEV - 1:
                rdma_wq = pltpu.make_async_remote_copy(
                    src_ref=comm_wq.at[h],
                    dst_ref=comm_wq.at[h + 1],
                    send_sem=send_sems.at[h, 0],
                    recv_sem=recv_sems.at[h, 0],
                    device_id=(right,),
                    device_id_type=pl.DeviceIdType.MESH,
                )
                rdma_wo = pltpu.make_async_remote_copy(
                    src_ref=comm_wo.at[h],
                    dst_ref=comm_wo.at[h + 1],
                    send_sem=send_sems.at[h, 1],
                    recv_sem=recv_sems.at[h, 1],
                    device_id=(right,),
                    device_id_type=pl.DeviceIdType.MESH,
                )
                rdma_wq.start()
                rdma_wo.start()

            origin = lax.rem(me + N_DEV - h, N_DEV)
            wq_blk = comm_wq[h]
            wo_blk = comm_wo[h]
            for b in range(B_LOC):
                q_all = jnp.dot(
                    x_ref[b], wq_blk, preferred_element_type=jnp.float32
                )
                ctx_parts = []
                for hh in range(H_LOC):
                    ih = origin * H_LOC + hh
                    k = kt_ref[b, pl.ds(ih, 1)].reshape(SQ, DH)
                    v = vt_ref[b, pl.ds(ih, 1)].reshape(SQ, DH)
                    q = q_all[:, hh * DH:(hh + 1) * DH]
                    s = lax.dot_general(
                        q, k, (((1,), (1,)), ((), ())),
                        preferred_element_type=jnp.float32,
                    ) * SCALE
                    m = jnp.max(s, axis=-1, keepdims=True)
                    w = jnp.exp(s - m)
                    w = w / jnp.sum(w, axis=-1, keepdims=True)
                    ctx_parts.append(
                        jnp.dot(w, v, preferred_element_type=jnp.float32)
                    )
                ctx = jnp.concatenate(ctx_parts, axis=1)
                contrib = jnp.dot(
                    ctx, wo_blk, preferred_element_type=jnp.float32
                )
                if h == 0:
                    out_ref[b] = contrib
                else:
                    out_ref[b] = out_ref[b] + contrib

            if h < N_DEV - 1:
                rdma_wq.wait()
                rdma_wo.wait()

    wq_s = Wq.shape
    wo_s = Wo.shape
    return pl.pallas_call(
        body,
        out_shape=jax.ShapeDtypeStruct(x.shape, jnp.float32),
        in_specs=[pl.BlockSpec(memory_space=pltpu.VMEM)] * 5,
        out_specs=pl.BlockSpec(memory_space=pltpu.VMEM),
        scratch_shapes=[
            pltpu.VMEM((N_DEV,) + wq_s, jnp.float32),
            pltpu.VMEM((N_DEV,) + wo_s, jnp.float32),
            pltpu.SemaphoreType.DMA((N_DEV - 1, 2)),
            pltpu.SemaphoreType.DMA((N_DEV - 1, 2)),
        ],
        compiler_params=pltpu.CompilerParams(collective_id=0),
    )(x, Wq, K_t, V_t, Wo)


# device time: 32383 ns/iter; 1.5128x vs baseline; 1.5128x over previous
import jax
import jax.numpy as jnp
from jax import lax
from jax.experimental import pallas as pl
from jax.experimental.pallas import tpu as pltpu

N_DEV = 4
B_LOC = 2
H_LOC = 4
H_HALF = 2
SQ = 128
DH = 64
DHH = H_HALF * DH
SCALE = 0.125


def kernel(x, Wq, K_ext, V_ext, Wo):
    my = lax.axis_index("i")
    K_my = lax.dynamic_slice_in_dim(K_ext, my * B_LOC, B_LOC, axis=0)
    V_my = lax.dynamic_slice_in_dim(V_ext, my * B_LOC, B_LOC, axis=0)
    K_t = jnp.transpose(K_my, (0, 2, 1, 3))
    V_t = jnp.transpose(V_my, (0, 2, 1, 3))

    def body(x_ref, wq_ref, kt_ref, vt_ref, wo_ref, out_ref,
             wq_cw, wo_cw, wq_ccw, wo_ccw, send_sems, recv_sems):
        me = lax.axis_index("i")
        right = lax.rem(me + 1, N_DEV)
        left = lax.rem(me + N_DEV - 1, N_DEV)

        barrier_sem = pltpu.get_barrier_semaphore()
        for nbr in (left, right):
            pl.semaphore_signal(
                barrier_sem, inc=1,
                device_id=(nbr,), device_id_type=pl.DeviceIdType.MESH,
            )
        pl.semaphore_wait(barrier_sem, 2)

        wq_cw[0] = wq_ref[:, :DHH]
        wo_cw[0] = wo_ref[:DHH, :]
        wq_ccw[0] = wq_ref[:, DHH:]
        wo_ccw[0] = wo_ref[DHH:, :]

        def issue(h):
            rs = []
            streams = (
                (wq_cw, right, 0), (wo_cw, right, 1),
                (wq_ccw, left, 2), (wo_ccw, left, 3),
            )
            for buf, nbr, si in streams:
                r = pltpu.make_async_remote_copy(
                    src_ref=buf.at[h - 1],
                    dst_ref=buf.at[h],
                    send_sem=send_sems.at[h - 1, si],
                    recv_sem=recv_sems.at[h - 1, si],
                    device_id=(nbr,),
                    device_id_type=pl.DeviceIdType.MESH,
                )
                r.start()
                rs.append(r)
            return rs

        def attend_half(b, q_half, o, head_off, wo_half):
            ctx_parts = []
            for hh in range(H_HALF):
                ih = o * H_LOC + head_off + hh
                k = kt_ref[b, pl.ds(ih, 1)].reshape(SQ, DH)
                v = vt_ref[b, pl.ds(ih, 1)].reshape(SQ, DH)
                q = q_half[:, hh * DH:(hh + 1) * DH]
                s = lax.dot_general(
                    q, k, (((1,), (1,)), ((), ())),
                    preferred_element_type=jnp.float32,
                ) * SCALE
                m = jnp.max(s, axis=-1, keepdims=True)
                w = jnp.exp(s - m)
                w = w / jnp.sum(w, axis=-1, keepdims=True)
                ctx_parts.append(
                    jnp.dot(w, v, preferred_element_type=jnp.float32)
                )
            ctx = jnp.concatenate(ctx_parts, axis=1)
            return jnp.dot(ctx, wo_half, preferred_element_type=jnp.float32)

        def compute(h):
            o_cw = lax.rem(me + N_DEV - h, N_DEV)
            o_ccw = lax.rem(me + h, N_DEV)
            for b in range(B_LOC):
                xb = x_ref[b]
                q_cw = jnp.dot(
                    xb, wq_cw[h], preferred_element_type=jnp.float32
                )
                q_ccw = jnp.dot(
                    xb, wq_ccw[h], preferred_element_type=jnp.float32
                )
                c = (attend_half(b, q_cw, o_cw, 0, wo_cw[h])
                     + attend_half(b, q_ccw, o_ccw, H_HALF, wo_ccw[h]))
                if h == 0:
                    out_ref[b] = c
                else:
                    out_ref[b] = out_ref[b] + c

        all_rdmas = issue(1)
        compute(0)
        for h in range(1, N_DEV):
            for r in all_rdmas[-4:]:
                r.wait_recv()
            if h < N_DEV - 1:
                all_rdmas += issue(h + 1)
            compute(h)
        for r in all_rdmas:
            r.wait_send()

    wq_s = Wq.shape
    wo_s = Wo.shape
    return pl.pallas_call(
        body,
        out_shape=jax.ShapeDtypeStruct(x.shape, jnp.float32),
        in_specs=[pl.BlockSpec(memory_space=pltpu.VMEM)] * 5,
        out_specs=pl.BlockSpec(memory_space=pltpu.VMEM),
        scratch_shapes=[
            pltpu.VMEM((N_DEV, wq_s[0], DHH), jnp.float32),
            pltpu.VMEM((N_DEV, DHH, wo_s[1]), jnp.float32),
            pltpu.VMEM((N_DEV, wq_s[0], DHH), jnp.float32),
            pltpu.VMEM((N_DEV, DHH, wo_s[1]), jnp.float32),
            pltpu.SemaphoreType.DMA((N_DEV - 1, 4)),
            pltpu.SemaphoreType.DMA((N_DEV - 1, 4)),
        ],
        compiler_params=pltpu.CompilerParams(collective_id=0),
    )(x, Wq, K_t, V_t, Wo)


# device time: 29096 ns/iter; 1.6837x vs baseline; 1.1130x over previous
import jax
import jax.numpy as jnp
from jax import lax
from jax.experimental import pallas as pl
from jax.experimental.pallas import tpu as pltpu

N_DEV = 4
B_LOC = 2
H_LOC = 4
H_HALF = 2
SQ = 128
DH = 64
DHH = H_HALF * DH
SCALE = 0.125


def kernel(x, Wq, K_ext, V_ext, Wo):
    my = lax.axis_index("i")
    K_my = lax.dynamic_slice_in_dim(K_ext, my * B_LOC, B_LOC, axis=0)
    V_my = lax.dynamic_slice_in_dim(V_ext, my * B_LOC, B_LOC, axis=0)
    K_t = jnp.transpose(K_my, (0, 2, 1, 3))
    V_t = jnp.transpose(V_my, (0, 2, 1, 3))

    def body(x_ref, wq_ref, kt_ref, vt_ref, wo_ref, out_ref,
             snd_wq_cw, snd_wq_ccw,
             l_wq_cw, l_wo_cw, l_wq_ccw, l_wo_ccw,
             r_wq_cw, r_wo_cw, r_wq_ccw, r_wo_ccw,
             o_wq_cw, o_wo_cw, o_wq_ccw, o_wo_ccw,
             send_sems, recv_sems):
        me = lax.axis_index("i")
        right = lax.rem(me + 1, N_DEV)
        left = lax.rem(me + N_DEV - 1, N_DEV)
        opp = lax.rem(me + 2, N_DEV)

        barrier_sem = pltpu.get_barrier_semaphore()
        for nbr in (left, right):
            pl.semaphore_signal(
                barrier_sem, inc=1,
                device_id=(nbr,), device_id_type=pl.DeviceIdType.MESH,
            )
        pl.semaphore_wait(barrier_sem, 2)

        snd_wq_cw[...] = wq_ref[:, :DHH]
        snd_wq_ccw[...] = wq_ref[:, DHH:]

        def mk(src, dst, idx, nbr):
            return pltpu.make_async_remote_copy(
                src_ref=src, dst_ref=dst,
                send_sem=send_sems.at[idx], recv_sem=recv_sems.at[idx],
                device_id=(nbr,), device_id_type=pl.DeviceIdType.MESH,
            )

        wo_cw_src = wo_ref.at[pl.ds(0, DHH)]
        wo_ccw_src = wo_ref.at[pl.ds(DHH, DHH)]

        hop1 = [
            mk(snd_wq_cw, l_wq_cw, 0, right),
            mk(wo_cw_src, l_wo_cw, 1, right),
            mk(snd_wq_ccw, l_wq_ccw, 2, right),
            mk(wo_ccw_src, l_wo_ccw, 3, right),
            mk(snd_wq_ccw, r_wq_ccw, 4, left),
            mk(wo_ccw_src, r_wo_ccw, 5, left),
            mk(snd_wq_cw, r_wq_cw, 6, left),
            mk(wo_cw_src, r_wo_cw, 7, left),
        ]
        for r in hop1:
            r.start()

        def attend_half(b, q_half, o, head_off, wo_half):
            ctx_parts = []
            for hh in range(H_HALF):
                ih = o * H_LOC + head_off + hh
                k = kt_ref[b, pl.ds(ih, 1)].reshape(SQ, DH)
                v = vt_ref[b, pl.ds(ih, 1)].reshape(SQ, DH)
                q = q_half[:, hh * DH:(hh + 1) * DH]
                s = lax.dot_general(
                    q, k, (((1,), (1,)), ((), ())),
                    preferred_element_type=jnp.float32,
                ) * SCALE
                m = jnp.max(s, axis=-1, keepdims=True)
                w = jnp.exp(s - m)
                w = w / jnp.sum(w, axis=-1, keepdims=True)
                ctx_parts.append(
                    jnp.dot(w, v, preferred_element_type=jnp.float32)
                )
            ctx = jnp.concatenate(ctx_parts, axis=1)
            return jnp.dot(ctx, wo_half, preferred_element_type=jnp.float32)

        def compute_half(origin, head_off, wq_half_ref, wo_half_ref,
                         init=False):
            wq_half = wq_half_ref[...]
            wo_half = wo_half_ref[...]
            for b in range(B_LOC):
                q = jnp.dot(
                    x_ref[b], wq_half, preferred_element_type=jnp.float32
                )
                c = attend_half(b, q, origin, head_off, wo_half)
                if init:
                    out_ref[b] = c
                else:
                    out_ref[b] = out_ref[b] + c

        compute_half(me, 0, snd_wq_cw, wo_cw_src, init=True)
        compute_half(me, H_HALF, snd_wq_ccw, wo_ccw_src)

        hop1[0].wait_recv()
        hop1[1].wait_recv()
        hop2 = [
            mk(l_wq_cw, o_wq_cw, 8, right),
            mk(l_wo_cw, o_wo_cw, 9, right),
        ]
        hop1[4].wait_recv()
        hop1[5].wait_recv()
        hop2 += [
            mk(r_wq_ccw, o_wq_ccw, 10, left),
            mk(r_wo_ccw, o_wo_ccw, 11, left),
        ]
        for r in hop2:
            r.start()

        compute_half(left, 0, l_wq_cw, l_wo_cw)
        compute_half(right, H_HALF, r_wq_ccw, r_wo_ccw)

        hop1[2].wait_recv()
        hop1[3].wait_recv()
        compute_half(left, H_HALF, l_wq_ccw, l_wo_ccw)

        hop1[6].wait_recv()
        hop1[7].wait_recv()
        compute_half(right, 0, r_wq_cw, r_wo_cw)

        hop2[0].wait_recv()
        hop2[1].wait_recv()
        compute_half(opp, 0, o_wq_cw, o_wo_cw)

        hop2[2].wait_recv()
        hop2[3].wait_recv()
        compute_half(opp, H_HALF, o_wq_ccw, o_wo_ccw)

        for r in hop1 + hop2:
            r.wait_send()

    wq_half_t = pltpu.VMEM((Wq.shape[0], DHH), jnp.float32)
    wo_half_t = pltpu.VMEM((DHH, Wo.shape[1]), jnp.float32)
    return pl.pallas_call(
        body,
        out_shape=jax.ShapeDtypeStruct(x.shape, jnp.float32),
        in_specs=[pl.BlockSpec(memory_space=pltpu.VMEM)] * 5,
        out_specs=pl.BlockSpec(memory_space=pltpu.VMEM),
        scratch_shapes=[
            wq_half_t, wq_half_t,
            wq_half_t, wo_half_t, wq_half_t, wo_half_t,
            wq_half_t, wo_half_t, wq_half_t, wo_half_t,
            wq_half_t, wo_half_t, wq_half_t, wo_half_t,
            pltpu.SemaphoreType.DMA((12,)),
            pltpu.SemaphoreType.DMA((12,)),
        ],
        compiler_params=pltpu.CompilerParams(collective_id=0),
    )(x, Wq, K_t, V_t, Wo)
